# Initial kernel scaffold; baseline (speedup 1.0000x reference)
import jax
import jax.numpy as jnp
from jax.experimental import pallas as pl


def kernel(x, edge_index, W1, b1, Wrel, brel, Wroot, W2, b2, Wih, Whh, bih, bhh, Wout, bout):
    raise NotImplementedError("write your pallas kernel here")


# speedup vs baseline: 1.0451x; 1.0451x over previous
import functools
import math

import jax
import jax.numpy as jnp
from jax.experimental import pallas as pl

N = 10000
E = 320000
T = 8
F_IN = 128
CH = 16
HID = 16
K_POOL = int(math.ceil(0.8 * N))


def _mm_body(a_ref, w_ref, b_ref, o_ref, *, act):
    acc = jnp.dot(a_ref[...], w_ref[...], preferred_element_type=jnp.float32)
    acc = acc + b_ref[...]
    if act == "relu":
        acc = jnp.maximum(acc, 0.0)
    o_ref[...] = acc


def _mm(a, w, b, act="none", block_m=None):
    m, k = a.shape
    n = w.shape[1]
    if block_m is None:
        block_m = m
    grid = (m // block_m,)
    b2 = jnp.reshape(b, (1, n))
    return pl.pallas_call(
        functools.partial(_mm_body, act=act),
        grid=grid,
        in_specs=[
            pl.BlockSpec((block_m, k), lambda i: (i, 0)),
            pl.BlockSpec((k, n), lambda i: (0, 0)),
            pl.BlockSpec((1, n), lambda i: (0, 0)),
        ],
        out_specs=pl.BlockSpec((block_m, n), lambda i: (i, 0)),
        out_shape=jax.ShapeDtypeStruct((m, n), jnp.float32),
    )(a, w, b2)


def _bias_relu_body(x_ref, b_ref, o_ref):
    o_ref[...] = jnp.maximum(x_ref[...] + b_ref[...], 0.0)


def _bias_relu(x, b):
    m, n = x.shape
    return pl.pallas_call(
        _bias_relu_body,
        in_specs=[
            pl.BlockSpec((m, n), lambda: (0, 0)),
            pl.BlockSpec((1, n), lambda: (0, 0)),
        ],
        out_specs=pl.BlockSpec((m, n), lambda: (0, 0)),
        out_shape=jax.ShapeDtypeStruct((m, n), jnp.float32),
    )(x, jnp.reshape(b, (1, n)))


def _lstm_body(seq_ref, wih_t_ref, whh_t_ref, bias_ref, wout_t_ref, bout_ref,
               o_ref):
    def step(t, hc):
        h, c = hc
        xt = seq_ref[pl.ds(t, 1), :]
        g = (jnp.dot(xt, wih_t_ref[...], preferred_element_type=jnp.float32)
             + jnp.dot(h, whh_t_ref[...], preferred_element_type=jnp.float32)
             + bias_ref[...])
        i = jax.nn.sigmoid(g[:, 0 * HID:1 * HID])
        f = jax.nn.sigmoid(g[:, 1 * HID:2 * HID])
        gg = jnp.tanh(g[:, 2 * HID:3 * HID])
        o = jax.nn.sigmoid(g[:, 3 * HID:4 * HID])
        c = f * c + i * gg
        h = o * jnp.tanh(c)
        return (h, c)

    h0 = jnp.zeros((1, HID), dtype=jnp.float32)
    c0 = jnp.zeros((1, HID), dtype=jnp.float32)
    h, _ = jax.lax.fori_loop(0, T, step, (h0, c0))
    pred = jnp.dot(h, wout_t_ref[...], preferred_element_type=jnp.float32)
    o_ref[...] = jax.nn.sigmoid(pred + bout_ref[...])


def _lstm_head(seq, Wih, Whh, bih, bhh, Wout, bout):
    bias = jnp.reshape(bih + bhh, (1, 4 * HID))
    return pl.pallas_call(
        _lstm_body,
        in_specs=[pl.BlockSpec(s, lambda: tuple(0 for _ in s)) for s in
                  [(T, HID), (HID, 4 * HID), (HID, 4 * HID), (1, 4 * HID),
                   (HID, 1), (1, 1)]],
        out_specs=pl.BlockSpec((1, 1), lambda: (0, 0)),
        out_shape=jax.ShapeDtypeStruct((1, 1), jnp.float32),
    )(seq, Wih.T, Whh.T, bias, Wout.T, jnp.reshape(bout, (1, 1)))


def _scatter_conv(xw, src2, dst2, w2, n):
    deg = jax.ops.segment_sum(w2, dst2, num_segments=n)
    dinv = jnp.where(deg > 0, jax.lax.rsqrt(jnp.where(deg > 0, deg, 1.0)), 0.0)
    norm = dinv[src2] * w2 * dinv[dst2]
    return jax.ops.segment_sum(xw[src2] * norm[:, None], dst2, num_segments=n)


def kernel(x, edge_index, W1, b1, Wrel, brel, Wroot, W2, b2, Wih, Whh, bih,
           bhh, Wout, bout):
    xw_all = _mm(x.reshape(T * N, F_IN), W1, jnp.zeros((CH,), jnp.float32),
                 block_m=8000)
    xw_all = xw_all.reshape(T, N, CH)

    loop_n = jnp.arange(N, dtype=jnp.int32)
    ones_n = jnp.ones((N,), jnp.float32)
    score_w = jnp.concatenate([Wrel, Wroot], axis=0)
    loop_k = jnp.arange(K_POOL, dtype=jnp.int32)
    ones_k = jnp.ones((K_POOL,), jnp.float32)

    embs = []
    for t in range(T):
        src = edge_index[t, 0]
        dst = edge_index[t, 1]

        src2 = jnp.concatenate([src, loop_n])
        dst2 = jnp.concatenate([dst, loop_n])
        w2 = jnp.ones((E + N,), jnp.float32)
        conv1 = _scatter_conv(xw_all[t], src2, dst2, w2, N)
        h = _bias_relu(conv1, b1)

        agg = jax.ops.segment_sum(h[src], dst, num_segments=N)
        score = _mm(jnp.concatenate([agg, h], axis=1), score_w, brel)
        score = score.reshape(-1)
        vals, perm = jax.lax.top_k(score, K_POOL)
        hp = h[perm] * jnp.tanh(vals)[:, None]

        new_idx = jnp.full((N,), -1, dtype=jnp.int32).at[perm].set(loop_k)
        es = new_idx[src]
        ed = new_idx[dst]
        valid = (es >= 0) & (ed >= 0)
        es = jnp.where(valid, es, 0)
        ed = jnp.where(valid, ed, 0)
        w = valid.astype(jnp.float32)

        xw2 = _mm(hp, W2, jnp.zeros((HID,), jnp.float32))
        src2b = jnp.concatenate([es, loop_k])
        dst2b = jnp.concatenate([ed, loop_k])
        w2b = jnp.concatenate([w, ones_k])
        conv2 = _scatter_conv(xw2, src2b, dst2b, w2b, K_POOL)
        h2 = _bias_relu(conv2, b2)
        embs.append(jnp.mean(h2, axis=0))

    seq = jnp.stack(embs, axis=0)
    return _lstm_head(seq, Wih, Whh, bih, bhh, Wout, bout)


# speedup vs baseline: 2.1806x; 2.0865x over previous
import functools
import math

import jax
import jax.numpy as jnp
from jax.experimental import pallas as pl

N = 10000
E = 320000
T = 8
F_IN = 128
CH = 16
HID = 16
K_POOL = int(math.ceil(0.8 * N))


def _mm_body(a_ref, w_ref, b_ref, o_ref, *, act):
    acc = jnp.dot(a_ref[...], w_ref[...], preferred_element_type=jnp.float32)
    acc = acc + b_ref[...]
    if act == "relu":
        acc = jnp.maximum(acc, 0.0)
    o_ref[...] = acc


def _mm(a, w, b, act="none", block_m=None):
    m, k = a.shape
    n = w.shape[1]
    if block_m is None:
        block_m = m
    grid = (m // block_m,)
    b2 = jnp.reshape(b, (1, n))
    return pl.pallas_call(
        functools.partial(_mm_body, act=act),
        grid=grid,
        in_specs=[
            pl.BlockSpec((block_m, k), lambda i: (i, 0)),
            pl.BlockSpec((k, n), lambda i: (0, 0)),
            pl.BlockSpec((1, n), lambda i: (0, 0)),
        ],
        out_specs=pl.BlockSpec((block_m, n), lambda i: (i, 0)),
        out_shape=jax.ShapeDtypeStruct((m, n), jnp.float32),
    )(a, w, b2)


def _conv_epilogue_body(s_ref, y_ref, d_ref, b_ref, o_ref):
    o_ref[...] = jnp.maximum(
        (s_ref[...] + y_ref[...]) * d_ref[...] + b_ref[...], 0.0)


def _conv_epilogue(s, y, dinv, b):
    m, n = s.shape
    return pl.pallas_call(
        _conv_epilogue_body,
        in_specs=[
            pl.BlockSpec((m, n), lambda: (0, 0)),
            pl.BlockSpec((m, n), lambda: (0, 0)),
            pl.BlockSpec((m, 1), lambda: (0, 0)),
            pl.BlockSpec((1, n), lambda: (0, 0)),
        ],
        out_specs=pl.BlockSpec((m, n), lambda: (0, 0)),
        out_shape=jax.ShapeDtypeStruct((m, n), jnp.float32),
    )(s, y, jnp.reshape(dinv, (m, 1)), jnp.reshape(b, (1, n)))


def _lstm_body(seq_ref, wih_t_ref, whh_t_ref, bias_ref, wout_t_ref, bout_ref,
               o_ref):
    def step(t, hc):
        h, c = hc
        xt = seq_ref[pl.ds(t, 1), :]
        g = (jnp.dot(xt, wih_t_ref[...], preferred_element_type=jnp.float32)
             + jnp.dot(h, whh_t_ref[...], preferred_element_type=jnp.float32)
             + bias_ref[...])
        i = jax.nn.sigmoid(g[:, 0 * HID:1 * HID])
        f = jax.nn.sigmoid(g[:, 1 * HID:2 * HID])
        gg = jnp.tanh(g[:, 2 * HID:3 * HID])
        o = jax.nn.sigmoid(g[:, 3 * HID:4 * HID])
        c = f * c + i * gg
        h = o * jnp.tanh(c)
        return (h, c)

    h0 = jnp.zeros((1, HID), dtype=jnp.float32)
    c0 = jnp.zeros((1, HID), dtype=jnp.float32)
    h, _ = jax.lax.fori_loop(0, T, step, (h0, c0))
    pred = jnp.dot(h, wout_t_ref[...], preferred_element_type=jnp.float32)
    o_ref[...] = jax.nn.sigmoid(pred + bout_ref[...])


def _lstm_head(seq, Wih, Whh, bih, bhh, Wout, bout):
    bias = jnp.reshape(bih + bhh, (1, 4 * HID))
    return pl.pallas_call(
        _lstm_body,
        in_specs=[pl.BlockSpec(s, lambda: tuple(0 for _ in s)) for s in
                  [(T, HID), (HID, 4 * HID), (HID, 4 * HID), (1, 4 * HID),
                   (HID, 1), (1, 1)]],
        out_specs=pl.BlockSpec((1, 1), lambda: (0, 0)),
        out_shape=jax.ShapeDtypeStruct((1, 1), jnp.float32),
    )(seq, Wih.T, Whh.T, bias, Wout.T, jnp.reshape(bout, (1, 1)))


def kernel(x, edge_index, W1, b1, Wrel, brel, Wroot, W2, b2, Wih, Whh, bih,
           bhh, Wout, bout):
    xw_all = _mm(x.reshape(T * N, F_IN), W1, jnp.zeros((CH,), jnp.float32),
                 block_m=8000)
    xw_all = xw_all.reshape(T, N, CH)

    score_w = jnp.concatenate([Wrel, Wroot], axis=0)
    loop_k = jnp.arange(K_POOL, dtype=jnp.int32)
    ones_e = jnp.ones((E,), jnp.float32)

    embs = []
    for t in range(T):
        src = edge_index[t, 0]
        dst = edge_index[t, 1]

        deg = jax.ops.segment_sum(ones_e, dst, num_segments=N) + 1.0
        dinv = jax.lax.rsqrt(deg)
        y = xw_all[t] * dinv[:, None]
        s = jax.ops.segment_sum(y[src], dst, num_segments=N)
        h = _conv_epilogue(s, y, dinv, b1)

        agg = jax.ops.segment_sum(h[src], dst, num_segments=N)
        score = _mm(jnp.concatenate([agg, h], axis=1), score_w, brel)
        score = score.reshape(-1)
        vals, perm = jax.lax.top_k(score, K_POOL)
        hp = h[perm] * jnp.tanh(vals)[:, None]

        new_idx = jnp.full((N,), -1, dtype=jnp.int32).at[perm].set(loop_k)
        es = new_idx[src]
        ed = new_idx[dst]
        valid = (es >= 0) & (ed >= 0)
        es = jnp.where(valid, es, 0)
        ed = jnp.where(valid, ed, 0)
        w = valid.astype(jnp.float32)

        xw2 = _mm(hp, W2, jnp.zeros((HID,), jnp.float32))
        deg2 = jax.ops.segment_sum(w, ed, num_segments=K_POOL) + 1.0
        dinv2 = jax.lax.rsqrt(deg2)
        y2 = xw2 * dinv2[:, None]
        s2 = jax.ops.segment_sum(y2[es] * w[:, None], ed, num_segments=K_POOL)
        h2 = _conv_epilogue(s2, y2, dinv2, b2)
        embs.append(jnp.mean(h2, axis=0))

    seq = jnp.stack(embs, axis=0)
    return _lstm_head(seq, Wih, Whh, bih, bhh, Wout, bout)
